# Initial kernel scaffold; baseline (speedup 1.0000x reference)
import jax
import jax.numpy as jnp
from jax.experimental import pallas as pl


def kernel(x, edge_index, ratio, monomer_id, batch, task_id, gin_w1, gin_b1, gin_w2, gin_b2, qW, qb, kW, kb, vW, vb, oW1, ob1, oW2, ob2):
    raise NotImplementedError("write your pallas kernel here")


# speedup vs baseline: 3.3666x; 3.3666x over previous
import functools
import jax
import jax.numpy as jnp
from jax import lax
from jax.experimental import pallas as pl
from jax.experimental.pallas import tpu as pltpu
from jax.experimental.pallas import tpu_sc as plsc

N, E, H = 10000, 320000, 128
M = 2048
B = 512
NC, NS = 2, 16
NW = NC * NS
CHUNK = 80
EPW = E // NW
NCH_E = EPW // CHUNK
NP = 10240
RCH = NP // NW // CHUNK
MPW = M // NW

_mesh = plsc.VectorSubcoreMesh(core_axis_name="c", subcore_axis_name="s")


@functools.partial(
    pl.kernel,
    mesh=_mesh,
    compiler_params=pltpu.CompilerParams(needs_layout_passes=False),
    out_type=jax.ShapeDtypeStruct((2 * NP, H), jnp.float32),
    scratch_types=[
        pltpu.VMEM((CHUNK,), jnp.int32),
        pltpu.VMEM((CHUNK,), jnp.int32),
        pltpu.VMEM((CHUNK, H), jnp.float32),
        pltpu.VMEM_SHARED((NP, H), jnp.float32),
        pltpu.SemaphoreType.DMA,
    ],
)
def _edge_agg(x_hbm, src_hbm, dst_hbm, zeros_hbm, out_hbm,
              src_v, dst_v, rows_v, acc_sh, sem):
    cid = lax.axis_index("c")
    sid = lax.axis_index("s")
    rpt = NP // NS
    pltpu.sync_copy(zeros_hbm.at[pl.ds(sid * rpt, rpt)],
                    acc_sh.at[pl.ds(sid * rpt, rpt)])
    plsc.subcore_barrier()
    wid = sid * NC + cid
    base = wid * EPW

    def body(c, carry):
        off = base + c * CHUNK
        pltpu.sync_copy(src_hbm.at[pl.ds(off, CHUNK)], src_v)
        pltpu.sync_copy(dst_hbm.at[pl.ds(off, CHUNK)], dst_v)
        pltpu.async_copy(x_hbm.at[src_v], rows_v, sem).wait()
        pltpu.sync_copy(rows_v, acc_sh.at[dst_v], add=True)
        return carry

    lax.fori_loop(0, NCH_E, body, 0)
    plsc.subcore_barrier()
    pltpu.sync_copy(acc_sh.at[pl.ds(sid * rpt, rpt)],
                    out_hbm.at[pl.ds(cid * NP + sid * rpt, rpt)])


@functools.partial(
    pl.kernel,
    mesh=_mesh,
    compiler_params=pltpu.CompilerParams(needs_layout_passes=False),
    out_type=[
        jax.ShapeDtypeStruct((2 * M, H), jnp.float32),
        jax.ShapeDtypeStruct((M,), jnp.float32),
        jax.ShapeDtypeStruct((M,), jnp.float32),
        jax.ShapeDtypeStruct((M,), jnp.int32),
    ],
    scratch_types=[
        pltpu.VMEM((CHUNK, H), jnp.float32),
        pltpu.VMEM((CHUNK,), jnp.int32),
        pltpu.VMEM((NP,), jnp.float32),
        pltpu.VMEM((NP,), jnp.int32),
        pltpu.VMEM((M,), jnp.int32),
        pltpu.VMEM((M,), jnp.int32),
        pltpu.VMEM((MPW,), jnp.float32),
        pltpu.VMEM((MPW,), jnp.float32),
        pltpu.VMEM((MPW,), jnp.int32),
        pltpu.VMEM_SHARED((M, H), jnp.float32),
    ],
)
def _pool(xp_hbm, mono_hbm, ratio_hbm, batch_hbm, starts_hbm, ends_hbm,
          zeros_hbm, emb_hbm, cnt_hbm, frac_hbm, bi_hbm,
          rows_v, mi_v, ratio_v, batch_v, st_v, en_v, cb, fb, bb, acc_sh):
    cid = lax.axis_index("c")
    sid = lax.axis_index("s")
    wid = sid * NC + cid
    mpt = M // NS
    pltpu.sync_copy(zeros_hbm.at[pl.ds(sid * mpt, mpt)],
                    acc_sh.at[pl.ds(sid * mpt, mpt)])
    plsc.subcore_barrier()
    base = wid * (NP // NW)

    def body(c, carry):
        off = base + c * CHUNK
        pltpu.sync_copy(mono_hbm.at[pl.ds(off, CHUNK)], mi_v)
        pltpu.sync_copy(xp_hbm.at[pl.ds(off, CHUNK)], rows_v)
        pltpu.sync_copy(rows_v, acc_sh.at[mi_v], add=True)
        return carry

    lax.fori_loop(0, RCH, body, 0)

    pltpu.sync_copy(ratio_hbm, ratio_v)
    pltpu.sync_copy(batch_hbm, batch_v)
    pltpu.sync_copy(starts_hbm, st_v)
    pltpu.sync_copy(ends_hbm, en_v)
    for j in range(MPW // 16):
        m0 = wid * MPW + j * 16
        sv = st_v[pl.ds(m0, 16)]
        ev = en_v[pl.ds(m0, 16)]
        cnt = ev - sv
        bi = plsc.load_gather(batch_v, [sv])
        maxc = jnp.max(cnt)

        def mbody(t, acc):
            idx = sv + t
            msk = idx < ev
            vals = plsc.load_gather(ratio_v, [jnp.where(msk, idx, 0)])
            return jnp.minimum(acc, jnp.where(msk, vals, jnp.inf))

        acc = lax.fori_loop(0, maxc, mbody,
                            jnp.full((16,), jnp.inf, jnp.float32))
        pos = pl.ds(j * 16, 16)
        cb[pos] = cnt.astype(jnp.float32)
        fb[pos] = jnp.where(cnt > 0, acc, 0.0)
        bb[pos] = bi
    obase = wid * MPW
    pltpu.sync_copy(cb, cnt_hbm.at[pl.ds(obase, MPW)])
    pltpu.sync_copy(fb, frac_hbm.at[pl.ds(obase, MPW)])
    pltpu.sync_copy(bb, bi_hbm.at[pl.ds(obase, MPW)])
    plsc.subcore_barrier()
    pltpu.sync_copy(acc_sh.at[pl.ds(sid * mpt, mpt)],
                    emb_hbm.at[pl.ds(cid * M + sid * mpt, mpt)])


BLK = 640


def _gin_body(x_ref, a0_ref, a1_ref, w1_ref, b1_ref, w2_ref, b2_ref, o_ref):
    h = x_ref[...] + a0_ref[...] + a1_ref[...]
    h1 = jnp.maximum(
        jnp.dot(h, w1_ref[...], preferred_element_type=jnp.float32,
                precision=lax.Precision.HIGHEST)
        + b1_ref[...], 0.0)
    h2 = (jnp.dot(h1, w2_ref[...], preferred_element_type=jnp.float32,
                precision=lax.Precision.HIGHEST)
          + b2_ref[...])
    o_ref[...] = jnp.maximum(h2, 0.0)


def _gin_call(x, agg2, w1, b1, w2, b2):
    nb = NP // BLK
    return pl.pallas_call(
        _gin_body,
        grid=(nb,),
        in_specs=[
            pl.BlockSpec((BLK, H), lambda i: (i, 0)),
            pl.BlockSpec((BLK, H), lambda i: (i, 0)),
            pl.BlockSpec((BLK, H), lambda i: (i + NP // BLK, 0)),
            pl.BlockSpec((H, H), lambda i: (0, 0)),
            pl.BlockSpec((1, H), lambda i: (0, 0)),
            pl.BlockSpec((H, H), lambda i: (0, 0)),
            pl.BlockSpec((1, H), lambda i: (0, 0)),
        ],
        out_specs=pl.BlockSpec((BLK, H), lambda i: (i, 0)),
        out_shape=jax.ShapeDtypeStruct((NP, H), jnp.float32),
    )(x, agg2, agg2, w1, b1, w2, b2)


def _dotT(a, b):
    return lax.dot_general(a, b, (((0,), (0,)), ((), ())),
                           preferred_element_type=jnp.float32,
                           precision=lax.Precision.HIGHEST)


def _att_body(ep0, ep1, cnt_ref, frac_ref, bi_ref, qW, qb, kW, kb, vW, vb,
              oW1, ob1, w2r, o_ref):
    cnt = cnt_ref[...]
    emb = (ep0[...] + ep1[...]) / jnp.where(cnt > 0, cnt, 1.0)
    frac = frac_ref[...]
    q = (jnp.dot(emb, qW[...], preferred_element_type=jnp.float32,
                precision=lax.Precision.HIGHEST)
         + qb[...]) * frac
    k = (jnp.dot(emb, kW[...], preferred_element_type=jnp.float32,
                precision=lax.Precision.HIGHEST)
         + kb[...]) * frac
    v = jnp.dot(emb, vW[...], preferred_element_type=jnp.float32,
                precision=lax.Precision.HIGHEST) + vb[...]
    oT = (bi_ref[...] ==
          lax.broadcasted_iota(jnp.int32, (B, M), 0)).astype(jnp.float32)
    ks = jnp.dot(oT, k, preferred_element_type=jnp.float32,
                precision=lax.Precision.HIGHEST)
    energy = q * _dotT(oT, ks) * jnp.float32(H) ** -0.5
    bvals = lax.broadcasted_iota(jnp.int32, (B, H), 0).astype(jnp.float32)
    bi_bc = _dotT(oT, bvals)
    NEG = jnp.float32(-3e38)
    m = energy
    d = 1
    while d < M:
        m_dn = jnp.concatenate(
            [jnp.full((d, H), NEG, jnp.float32), m[:-d]], axis=0)
        b_dn = jnp.concatenate(
            [jnp.full((d, H), -1.0, jnp.float32), bi_bc[:-d]], axis=0)
        m_up = jnp.concatenate(
            [m[d:], jnp.full((d, H), NEG, jnp.float32)], axis=0)
        b_up = jnp.concatenate(
            [bi_bc[d:], jnp.full((d, H), -1.0, jnp.float32)], axis=0)
        m = jnp.maximum(m, jnp.maximum(
            jnp.where(b_dn == bi_bc, m_dn, NEG),
            jnp.where(b_up == bi_bc, m_up, NEG)))
        d *= 2
    e = jnp.exp(energy - m)
    den = jnp.dot(oT, e, preferred_element_type=jnp.float32,
                precision=lax.Precision.HIGHEST)
    att = e / (_dotT(oT, den) + 1e-16)
    pol = jnp.dot(oT, v * att, preferred_element_type=jnp.float32,
                precision=lax.Precision.HIGHEST)
    r1 = jnp.maximum(
        jnp.dot(pol, oW1[...], preferred_element_type=jnp.float32,
                precision=lax.Precision.HIGHEST)
        + ob1[...], 0.0)
    o_ref[...] = jnp.sum(r1 * w2r[...], axis=1)[None, :]


def _att_call(embp, cnt_bc, frac_bc, bi_row, qW, qb, kW, kb, vW, vb,
              oW1, ob1, w2r):
    return pl.pallas_call(
        _att_body,
        grid=(1,),
        in_specs=[
            pl.BlockSpec((M, H), lambda i: (0, 0)),
            pl.BlockSpec((M, H), lambda i: (1, 0)),
            pl.BlockSpec((M, H), lambda i: (0, 0)),
            pl.BlockSpec((M, H), lambda i: (0, 0)),
            pl.BlockSpec((1, M), lambda i: (0, 0)),
            pl.BlockSpec((H, H), lambda i: (0, 0)),
            pl.BlockSpec((1, H), lambda i: (0, 0)),
            pl.BlockSpec((H, H), lambda i: (0, 0)),
            pl.BlockSpec((1, H), lambda i: (0, 0)),
            pl.BlockSpec((H, H), lambda i: (0, 0)),
            pl.BlockSpec((1, H), lambda i: (0, 0)),
            pl.BlockSpec((H, H), lambda i: (0, 0)),
            pl.BlockSpec((1, H), lambda i: (0, 0)),
            pl.BlockSpec((1, H), lambda i: (0, 0)),
        ],
        out_specs=pl.BlockSpec((1, B), lambda i: (0, 0)),
        out_shape=jax.ShapeDtypeStruct((1, B), jnp.float32),
    )(embp, embp, cnt_bc, frac_bc, bi_row, qW, qb, kW, kb, vW, vb,
      oW1, ob1, w2r)


def kernel(x, edge_index, ratio, monomer_id, batch, task_id,
           gin_w1, gin_b1, gin_w2, gin_b2, qW, qb, kW, kb, vW, vb,
           oW1, ob1, oW2, ob2):
    perm = jnp.argsort(edge_index[1], stable=True)
    src = edge_index[0][perm]
    dst = edge_index[1][perm]
    zeros_nh = jnp.zeros((NP, H), jnp.float32)
    x = jnp.concatenate([x, jnp.zeros((NP - N, H), jnp.float32)], axis=0)
    for i in range(3):
        agg2 = _edge_agg(x, src, dst, zeros_nh)
        x = _gin_call(x, agg2, gin_w1[i], gin_b1[i].reshape(1, H),
                      gin_w2[i], gin_b2[i].reshape(1, H))
    change = jnp.concatenate(
        [jnp.zeros((1,), jnp.int32),
         (monomer_id[1:] != monomer_id[:-1]).astype(jnp.int32)])
    monomer = jnp.cumsum(change, dtype=jnp.int32)
    starts_all = jnp.searchsorted(
        monomer, jnp.arange(M + 1, dtype=jnp.int32)).astype(jnp.int32)
    starts = starts_all[:M]
    ends = starts_all[1:]
    xp = jnp.concatenate([x[:N], jnp.zeros((NP - N, H), jnp.float32)],
                         axis=0)
    monop = jnp.concatenate([monomer, jnp.full((NP - N,), M - 1, jnp.int32)])
    ratiop = jnp.concatenate([ratio, jnp.zeros((NP - N,), jnp.float32)])
    batchp = jnp.concatenate([batch, jnp.full((NP - N,), B, jnp.int32)])
    embp, cnt, frac, bi = _pool(xp, monop, ratiop, batchp, starts, ends,
                                jnp.zeros((M, H), jnp.float32))
    cnt_bc = jnp.broadcast_to(cnt[:, None], (M, H))
    frac_bc = jnp.broadcast_to(frac[:, None], (M, H))
    bi_row = bi[None, :]
    res = _att_call(embp, cnt_bc, frac_bc, bi_row, qW, qb.reshape(1, H),
                    kW, kb.reshape(1, H), vW, vb.reshape(1, H), oW1,
                    ob1.reshape(1, H), oW2.reshape(1, H))
    return res[0] + ob2[0]
